# baseline (speedup 1.0000x reference)
import jax
import jax.numpy as jnp
from jax.experimental import pallas as pl
from jax.experimental.pallas import tpu as pltpu

N = 10000
NFEAT = 128
NEMB = 128
BM = 512


def _gcn_kernel(x_ref, w_ref, adj_ref, b_ref, out_ref, support_ref):
    i = pl.program_id(0)

    @pl.when(i == 0)
    def _():
        support_ref[...] = jnp.dot(
            x_ref[...], w_ref[...], preferred_element_type=jnp.float32
        )

    out_ref[...] = (
        jnp.dot(adj_ref[...], support_ref[...], preferred_element_type=jnp.float32)
        + b_ref[...]
    )


def kernel(x, adj, W, b):
    b2 = b.reshape(1, NEMB)
    grid = (pl.cdiv(N, BM),)
    return pl.pallas_call(
        _gcn_kernel,
        grid=grid,
        in_specs=[
            pl.BlockSpec((N, NFEAT), lambda i: (0, 0)),
            pl.BlockSpec((NFEAT, NEMB), lambda i: (0, 0)),
            pl.BlockSpec((BM, N), lambda i: (i, 0)),
            pl.BlockSpec((1, NEMB), lambda i: (0, 0)),
        ],
        out_specs=pl.BlockSpec((BM, NEMB), lambda i: (i, 0)),
        out_shape=jax.ShapeDtypeStruct((N, NEMB), jnp.float32),
        scratch_shapes=[pltpu.VMEM((N, NEMB), jnp.float32)],
    )(x, W, adj, b2)


# speedup vs baseline: 1.0041x; 1.0041x over previous
import jax
import jax.numpy as jnp
from jax.experimental import pallas as pl
from jax.experimental.pallas import tpu as pltpu

N = 10000
NFEAT = 128
NEMB = 128
BM = 200


def _gcn_kernel(x_ref, w_ref, adj_ref, b_ref, out_ref, support_ref):
    i = pl.program_id(0)

    @pl.when(i == 0)
    def _():
        support_ref[...] = jnp.dot(
            x_ref[...], w_ref[...], preferred_element_type=jnp.float32
        )

    out_ref[...] = (
        jnp.dot(adj_ref[...], support_ref[...], preferred_element_type=jnp.float32)
        + b_ref[...]
    )


def kernel(x, adj, W, b):
    b2 = b.reshape(1, NEMB)
    grid = (pl.cdiv(N, BM),)
    return pl.pallas_call(
        _gcn_kernel,
        grid=grid,
        in_specs=[
            pl.BlockSpec((N, NFEAT), lambda i: (0, 0)),
            pl.BlockSpec((NFEAT, NEMB), lambda i: (0, 0)),
            pl.BlockSpec((BM, N), lambda i: (i, 0)),
            pl.BlockSpec((1, NEMB), lambda i: (0, 0)),
        ],
        out_specs=pl.BlockSpec((BM, NEMB), lambda i: (i, 0)),
        out_shape=jax.ShapeDtypeStruct((N, NEMB), jnp.float32),
        scratch_shapes=[pltpu.VMEM((N, NEMB), jnp.float32)],
    )(x, W, adj, b2)


# speedup vs baseline: 1.0102x; 1.0060x over previous
import jax
import jax.numpy as jnp
from jax.experimental import pallas as pl
from jax.experimental.pallas import tpu as pltpu

N = 10000
NFEAT = 128
NEMB = 128
BM = 400


def _gcn_kernel(x_ref, w_ref, adj_ref, b_ref, out_ref, support_ref):
    i = pl.program_id(0)

    @pl.when(i == 0)
    def _():
        support_ref[...] = jnp.dot(
            x_ref[...], w_ref[...], preferred_element_type=jnp.float32
        )

    out_ref[...] = (
        jnp.dot(adj_ref[...], support_ref[...], preferred_element_type=jnp.float32)
        + b_ref[...]
    )


def kernel(x, adj, W, b):
    b2 = b.reshape(1, NEMB)
    grid = (pl.cdiv(N, BM),)
    return pl.pallas_call(
        _gcn_kernel,
        grid=grid,
        in_specs=[
            pl.BlockSpec((N, NFEAT), lambda i: (0, 0)),
            pl.BlockSpec((NFEAT, NEMB), lambda i: (0, 0)),
            pl.BlockSpec((BM, N), lambda i: (i, 0)),
            pl.BlockSpec((1, NEMB), lambda i: (0, 0)),
        ],
        out_specs=pl.BlockSpec((BM, NEMB), lambda i: (i, 0)),
        out_shape=jax.ShapeDtypeStruct((N, NEMB), jnp.float32),
        scratch_shapes=[pltpu.VMEM((N, NEMB), jnp.float32)],
    )(x, W, adj, b2)
